# Initial kernel scaffold; baseline (speedup 1.0000x reference)
import functools

import jax
import jax.numpy as jnp
from jax import lax
from jax.experimental import pallas as pl
from jax.experimental.pallas import tpu as pltpu

N_NODES = 100000
BLK = 4000


def _tc1_body(deg_ref, x_ref, dinv_ref, g0_ref):
    deg = deg_ref[0, :] + deg_ref[1, :] + 1.0
    dinv = lax.rsqrt(deg)
    dinv_ref[...] = dinv
    g0_ref[...] = x_ref[...] * dinv[:, None]


def _tc2_body(s1_ref, g0_ref, dinv_ref, w1_ref, b1_ref, g1_ref):
    s1 = s1_ref[0] + s1_ref[1]
    dinv = dinv_ref[...]
    a1 = (s1 + g0_ref[...]) * dinv[:, None]
    h1 = jnp.maximum(
        jnp.dot(a1, w1_ref[...], preferred_element_type=jnp.float32) + b1_ref[...],
        0.0,
    )
    g1_ref[...] = h1 * dinv[:, None]


def _tc3_body(s2_ref, g1_ref, dinv_ref, w2_ref, b2_ref, wfc_ref, bfc_ref, y_ref):
    s2 = jnp.concatenate([s2_ref[0], s2_ref[1]], axis=-1)
    dinv = dinv_ref[...]
    a2 = (s2 + g1_ref[...]) * dinv[:, None]
    h2 = jnp.maximum(
        jnp.dot(a2, w2_ref[...], preferred_element_type=jnp.float32) + b2_ref[...],
        0.0,
    )
    y_ref[...] = jnp.dot(h2, wfc_ref[...], preferred_element_type=jnp.float32) + bfc_ref[...]


def _full(shape):
    return pl.BlockSpec(shape, lambda i: tuple(0 for _ in shape))


def _tc1(deg_parts, x):
    n_blk = N_NODES // BLK
    return pl.pallas_call(
        _tc1_body,
        grid=(n_blk,),
        in_specs=[
            pl.BlockSpec((2, BLK), lambda i: (0, i)),
            pl.BlockSpec((BLK, 8), lambda i: (i, 0)),
        ],
        out_specs=[
            pl.BlockSpec((BLK,), lambda i: (i,)),
            pl.BlockSpec((BLK, 8), lambda i: (i, 0)),
        ],
        out_shape=[
            jax.ShapeDtypeStruct((N_NODES,), jnp.float32),
            jax.ShapeDtypeStruct((N_NODES, 8), jnp.float32),
        ],
    )(deg_parts, x)


def _tc2(s1_parts, g0, dinv, W1, b1):
    n_blk = N_NODES // BLK
    return pl.pallas_call(
        _tc2_body,
        grid=(n_blk,),
        in_specs=[
            pl.BlockSpec((2, BLK, 8), lambda i: (0, i, 0)),
            pl.BlockSpec((BLK, 8), lambda i: (i, 0)),
            pl.BlockSpec((BLK,), lambda i: (i,)),
            _full((8, 32)),
            _full((32,)),
        ],
        out_specs=pl.BlockSpec((BLK, 32), lambda i: (i, 0)),
        out_shape=jax.ShapeDtypeStruct((N_NODES, 32), jnp.float32),
    )(s1_parts, g0, dinv, W1, b1)


def _tc3(s2_parts, g1, dinv, W2, b2, Wfc, bfc):
    n_blk = N_NODES // BLK
    return pl.pallas_call(
        _tc3_body,
        grid=(n_blk,),
        in_specs=[
            pl.BlockSpec((2, BLK, 16), lambda i: (0, i, 0)),
            pl.BlockSpec((BLK, 32), lambda i: (i, 0)),
            pl.BlockSpec((BLK,), lambda i: (i,)),
            _full((32, 32)),
            _full((32,)),
            _full((32, 1)),
            _full((1,)),
        ],
        out_specs=pl.BlockSpec((BLK, 1), lambda i: (i, 0)),
        out_shape=jax.ShapeDtypeStruct((N_NODES, 1), jnp.float32),
    )(s2_parts, g1, dinv, W2, b2, Wfc, bfc)


@jax.jit
def kernel(x, edge_index, W1, b1, W2, b2, Wfc, bfc):
    src = edge_index[0]
    dst = edge_index[1]
    half = src.shape[0] // 2

    deg_parts = jnp.stack([
        jnp.zeros((N_NODES,), jnp.float32).at[dst[:half]].add(1.0),
        jnp.zeros((N_NODES,), jnp.float32).at[dst[half:]].add(1.0),
    ])
    dinv, g0 = _tc1(deg_parts, x)

    s1_parts = jnp.stack([
        jnp.zeros((N_NODES, 8), jnp.float32).at[dst[:half]].add(g0[src[:half]]),
        jnp.zeros((N_NODES, 8), jnp.float32).at[dst[half:]].add(g0[src[half:]]),
    ])
    g1 = _tc2(s1_parts, g0, dinv, W1, b1)

    g1v = g1.reshape(2 * N_NODES, 16)
    s2_parts = jnp.stack([
        jnp.zeros((N_NODES, 16), jnp.float32).at[dst].add(g1v[2 * src]),
        jnp.zeros((N_NODES, 16), jnp.float32).at[dst].add(g1v[2 * src + 1]),
    ])
    return _tc3(s2_parts, g1, dinv, W2, b2, Wfc, bfc)


# speedup vs baseline: 1.7680x; 1.7680x over previous
import functools

import jax
import jax.numpy as jnp
from jax import lax
from jax.experimental import pallas as pl
from jax.experimental.pallas import tpu as pltpu

N_NODES = 100000
BLK = 4000


def _tc1_body(deg0_ref, deg1_ref, x_ref, dinv_ref, g0_ref):
    deg = deg0_ref[...] + deg1_ref[...] + 1.0
    dinv = lax.rsqrt(deg)
    dinv_ref[...] = dinv
    g0_ref[...] = x_ref[...] * dinv


def _tc2_body(s1_ref, g0_ref, dinv_ref, w1_ref, b1_ref, g1_ref):
    s1 = s1_ref[0] + s1_ref[1]
    dinv = dinv_ref[...]
    a1 = (s1 + g0_ref[...]) * dinv
    h1 = jnp.maximum(
        jnp.dot(a1, w1_ref[...], preferred_element_type=jnp.float32) + b1_ref[...],
        0.0,
    )
    g1_ref[...] = h1 * dinv


def _tc3_body(s2_ref, g1_ref, dinv_ref, w2_ref, b2_ref, wfc_ref, bfc_ref, y_ref):
    s2 = jnp.concatenate([s2_ref[0], s2_ref[1]], axis=-1)
    dinv = dinv_ref[...]
    a2 = (s2 + g1_ref[...]) * dinv
    h2 = jnp.maximum(
        jnp.dot(a2, w2_ref[...], preferred_element_type=jnp.float32) + b2_ref[...],
        0.0,
    )
    y_ref[...] = jnp.dot(h2, wfc_ref[...], preferred_element_type=jnp.float32) + bfc_ref[...]


def _full(shape):
    return pl.BlockSpec(shape, lambda i: tuple(0 for _ in shape))


def _tc1(deg0, deg1, x):
    n_blk = N_NODES // BLK
    return pl.pallas_call(
        _tc1_body,
        grid=(n_blk,),
        in_specs=[
            pl.BlockSpec((BLK, 1), lambda i: (i, 0)),
            pl.BlockSpec((BLK, 1), lambda i: (i, 0)),
            pl.BlockSpec((BLK, 8), lambda i: (i, 0)),
        ],
        out_specs=[
            pl.BlockSpec((BLK, 1), lambda i: (i, 0)),
            pl.BlockSpec((BLK, 8), lambda i: (i, 0)),
        ],
        out_shape=[
            jax.ShapeDtypeStruct((N_NODES, 1), jnp.float32),
            jax.ShapeDtypeStruct((N_NODES, 8), jnp.float32),
        ],
    )(deg0, deg1, x)


def _tc2(s1_parts, g0, dinv, W1, b1):
    n_blk = N_NODES // BLK
    return pl.pallas_call(
        _tc2_body,
        grid=(n_blk,),
        in_specs=[
            pl.BlockSpec((2, BLK, 8), lambda i: (0, i, 0)),
            pl.BlockSpec((BLK, 8), lambda i: (i, 0)),
            pl.BlockSpec((BLK, 1), lambda i: (i, 0)),
            _full((8, 32)),
            _full((32,)),
        ],
        out_specs=pl.BlockSpec((BLK, 32), lambda i: (i, 0)),
        out_shape=jax.ShapeDtypeStruct((N_NODES, 32), jnp.float32),
    )(s1_parts, g0, dinv, W1, b1)


def _tc3(s2_parts, g1, dinv, W2, b2, Wfc, bfc):
    n_blk = N_NODES // BLK
    return pl.pallas_call(
        _tc3_body,
        grid=(n_blk,),
        in_specs=[
            pl.BlockSpec((2, BLK, 16), lambda i: (0, i, 0)),
            pl.BlockSpec((BLK, 32), lambda i: (i, 0)),
            pl.BlockSpec((BLK, 1), lambda i: (i, 0)),
            _full((32, 32)),
            _full((32,)),
            _full((32, 1)),
            _full((1,)),
        ],
        out_specs=pl.BlockSpec((BLK, 1), lambda i: (i, 0)),
        out_shape=jax.ShapeDtypeStruct((N_NODES, 1), jnp.float32),
    )(s2_parts, g1, dinv, W2, b2, Wfc, bfc)


@jax.jit
def kernel(x, edge_index, W1, b1, W2, b2, Wfc, bfc):
    src = edge_index[0]
    dst = edge_index[1]
    half = src.shape[0] // 2

    deg0 = jnp.zeros((N_NODES, 1), jnp.float32).at[dst[:half], 0].add(1.0)
    deg1 = jnp.zeros((N_NODES, 1), jnp.float32).at[dst[half:], 0].add(1.0)
    dinv, g0 = _tc1(deg0, deg1, x)

    s1_parts = jnp.stack([
        jnp.zeros((N_NODES, 8), jnp.float32).at[dst[:half]].add(g0[src[:half]]),
        jnp.zeros((N_NODES, 8), jnp.float32).at[dst[half:]].add(g0[src[half:]]),
    ])
    g1 = _tc2(s1_parts, g0, dinv, W1, b1)

    g1v = g1.reshape(2 * N_NODES, 16)
    s2_parts = jnp.stack([
        jnp.zeros((N_NODES, 16), jnp.float32).at[dst].add(g1v[2 * src]),
        jnp.zeros((N_NODES, 16), jnp.float32).at[dst].add(g1v[2 * src + 1]),
    ])
    return _tc3(s2_parts, g1, dinv, W2, b2, Wfc, bfc)


# speedup vs baseline: 2.5554x; 1.4454x over previous
import functools

import jax
import jax.numpy as jnp
from jax import lax
from jax.experimental import pallas as pl
from jax.experimental.pallas import tpu as pltpu
from jax.experimental.pallas import tpu_sc as plsc

N_NODES = 100000
N_EDGES = 1600000
BLK = 4000
LANE = 128
E_PAD = 1638400
ROWS = E_PAD // LANE
NTR = 96
NP = N_NODES + NTR
STRIPE = NP // 16
CH = 40
NB = 4
RPT = ROWS // 16

_mesh = plsc.VectorSubcoreMesh(core_axis_name="c", subcore_axis_name="s")
_sc_params = pltpu.CompilerParams(
    needs_layout_passes=False, use_tc_tiling_on_sc=False)


def _sc_pass(table, src2d, dst2d, params):
    @functools.partial(
        pl.kernel,
        out_type=jax.ShapeDtypeStruct((2 * N_NODES, 16), jnp.float32),
        mesh=_mesh,
        compiler_params=_sc_params,
        scratch_types=[
            pltpu.VMEM((CH, LANE), jnp.int32),
            pltpu.VMEM((CH, LANE), jnp.int32),
            pltpu.VMEM((CH, LANE), jnp.int32),
            [pltpu.VMEM((LANE, 16), jnp.float32) for _ in range(NB)],
            pltpu.VMEM((32,), jnp.int32),
            pltpu.VMEM_SHARED((NP, 16), jnp.float32),
            [pltpu.SemaphoreType.DMA for _ in range(NB)],
        ],
    )
    def pass_kernel(table_hbm, src_hbm, dst_hbm, par_hbm, out_hbm,
                    srcb, dstb, gidx, bufs, parb, acc, sems):
        c = lax.axis_index("c")
        s = lax.axis_index("s")
        zero16 = jnp.zeros((16,), jnp.float32)

        pltpu.sync_copy(par_hbm, parb)
        mulv = parb[pl.ds(0, 16)]
        coffv = jnp.broadcast_to(c, (16,)).astype(jnp.int32) * parb[pl.ds(16, 16)]

        def zfill(i, _):
            bufs[0][i] = zero16
            return 0

        lax.fori_loop(0, LANE, zfill, 0)
        base = s * STRIPE

        def zcopy(i, _):
            pltpu.sync_copy(bufs[0], acc.at[pl.ds(base + i * LANE, LANE)])
            return 0

        lax.fori_loop(0, STRIPE // LANE, zcopy, 0)
        pltpu.sync_copy(bufs[0].at[pl.ds(0, STRIPE % LANE)],
                        acc.at[pl.ds(base + (STRIPE // LANE) * LANE,
                                     STRIPE % LANE)])
        plsc.subcore_barrier()

        row0 = s * RPT

        def transform(j):
            for g in range(LANE // 16):
                v = srcb[j, pl.ds(g * 16, 16)]
                gidx[j, pl.ds(g * 16, 16)] = v * mulv + coffv

        for k in range(RPT // CH):
            pltpu.sync_copy(src_hbm.at[pl.ds(row0 + k * CH, CH)], srcb)
            pltpu.sync_copy(dst_hbm.at[pl.ds(row0 + k * CH, CH)], dstb)
            for b in range(NB):
                transform(b)
                pltpu.async_copy(table_hbm.at[gidx.at[b]], bufs[b], sems[b])

            def qstep(q, _):
                for b in range(NB):
                    row = q * NB + b
                    nxt = row + NB

                    @pl.when(nxt < CH)
                    def _():
                        transform(nxt)

                    pltpu.make_async_copy(
                        table_hbm.at[gidx.at[b]], bufs[b], sems[b]).wait()
                    pltpu.sync_copy(bufs[b], acc.at[dstb.at[row]], add=True)

                    @pl.when(nxt < CH)
                    def _():
                        pltpu.async_copy(
                            table_hbm.at[gidx.at[nxt]], bufs[b], sems[b])
                return 0

            lax.fori_loop(0, CH // NB, qstep, 0)

        plsc.subcore_barrier()
        last_valid = N_NODES - 15 * STRIPE

        @pl.when(s != 15)
        def _():
            pltpu.sync_copy(acc.at[pl.ds(base, STRIPE)],
                            out_hbm.at[pl.ds(c * N_NODES + base, STRIPE)])

        @pl.when(s == 15)
        def _():
            pltpu.sync_copy(
                acc.at[pl.ds(15 * STRIPE, last_valid)],
                out_hbm.at[pl.ds(c * N_NODES + 15 * STRIPE, last_valid)])

    return pass_kernel(table, src2d, dst2d, params)


def _tc1_body(deg_ref, x_ref, dinv_ref, g0_ref):
    deg = deg_ref[:, :1] + 1.0
    dinv = lax.rsqrt(deg)
    dinv_ref[...] = dinv
    g0 = x_ref[...] * dinv
    g0_ref[...] = jnp.concatenate(
        [g0, jnp.zeros((BLK, 8), jnp.float32)], axis=-1)


def _tc2_body(s1_ref, g0_ref, dinv_ref, w1_ref, b1_ref, g1_ref):
    dinv = dinv_ref[...]
    a1 = (s1_ref[...] + g0_ref[...]) * dinv
    h1 = jnp.maximum(
        jnp.dot(a1, w1_ref[...], preferred_element_type=jnp.float32)
        + b1_ref[...], 0.0)
    g1_ref[...] = h1 * dinv


def _tc3_body(s2_ref, g1_ref, dinv_ref, w2_ref, b2_ref, wfc_ref, bfc_ref,
              y_ref):
    s2 = jnp.concatenate([s2_ref[0], s2_ref[1]], axis=-1)
    dinv = dinv_ref[...]
    a2 = (s2 + g1_ref[...]) * dinv
    h2 = jnp.maximum(
        jnp.dot(a2, w2_ref[...], preferred_element_type=jnp.float32)
        + b2_ref[...], 0.0)
    y_ref[...] = (
        jnp.dot(h2, wfc_ref[...], preferred_element_type=jnp.float32)
        + bfc_ref[...])


def _full(shape):
    return pl.BlockSpec(shape, lambda i: tuple(0 for _ in shape))


def _row_spec(width):
    return pl.BlockSpec((BLK, width), lambda i: (i, 0))


def _tc1(deg, x):
    return pl.pallas_call(
        _tc1_body,
        grid=(N_NODES // BLK,),
        in_specs=[_row_spec(16), _row_spec(8)],
        out_specs=[_row_spec(1), _row_spec(16)],
        out_shape=[
            jax.ShapeDtypeStruct((N_NODES, 1), jnp.float32),
            jax.ShapeDtypeStruct((2 * N_NODES, 16), jnp.float32),
        ],
    )(deg, x)


def _tc2(s1, g0p, dinv, W1p, b1):
    return pl.pallas_call(
        _tc2_body,
        grid=(N_NODES // BLK,),
        in_specs=[
            _row_spec(16),
            _row_spec(16),
            _row_spec(1),
            _full((16, 32)),
            _full((32,)),
        ],
        out_specs=_row_spec(32),
        out_shape=jax.ShapeDtypeStruct((N_NODES, 32), jnp.float32),
    )(s1, g0p, dinv, W1p, b1)


def _tc3(s2_parts, g1, dinv, W2, b2, Wfc, bfc):
    return pl.pallas_call(
        _tc3_body,
        grid=(N_NODES // BLK,),
        in_specs=[
            pl.BlockSpec((2, BLK, 16), lambda i: (0, i, 0)),
            _row_spec(32),
            _row_spec(1),
            _full((32, 32)),
            _full((32,)),
            _full((32, 1)),
            _full((1,)),
        ],
        out_specs=_row_spec(1),
        out_shape=jax.ShapeDtypeStruct((N_NODES, 1), jnp.float32),
    )(s2_parts, g1, dinv, W2, b2, Wfc, bfc)


def _par(mul, cmul):
    return jnp.concatenate([
        jnp.full((16,), mul, jnp.int32),
        jnp.full((16,), cmul, jnp.int32),
    ])


@jax.jit
def kernel(x, edge_index, W1, b1, W2, b2, Wfc, bfc):
    pad = E_PAD - N_EDGES
    srcp = jnp.concatenate(
        [edge_index[0], jnp.zeros((pad,), jnp.int32)]).reshape(ROWS, LANE)
    dstp = jnp.concatenate(
        [edge_index[1], jnp.full((pad,), N_NODES, jnp.int32)]).reshape(
            ROWS, LANE)

    ones_t = jnp.ones((2 * N_NODES, 16), jnp.float32)
    deg_parts = _sc_pass(ones_t, srcp, dstp, _par(0, 0))
    dinv, g0p = _tc1(deg_parts[:N_NODES], x)

    s1 = _sc_pass(g0p, srcp, dstp, _par(1, 0))
    W1p = jnp.concatenate([W1, jnp.zeros((8, 32), jnp.float32)], axis=0)
    g1 = _tc2(s1[:N_NODES], g0p[:N_NODES], dinv, W1p, b1)

    s2 = _sc_pass(g1.reshape(2 * N_NODES, 16), srcp, dstp, _par(2, 1))
    return _tc3(s2.reshape(2, N_NODES, 16), g1, dinv, W2, b2, Wfc, bfc)


# speedup vs baseline: 18.3463x; 7.1793x over previous
import functools

import jax
import jax.numpy as jnp
from jax import lax
from jax.experimental import pallas as pl
from jax.experimental.pallas import tpu as pltpu
from jax.experimental.pallas import tpu_sc as plsc

N_NODES = 100000
N_EDGES = 1600000
BLK = 4000
LANE = 128
E_PAD = 1638400
ROWS = E_PAD // LANE
NTR = 96
NP = N_NODES + NTR
STRIPE = NP // 16
CH = 40
NB = 4
RPT = ROWS // 16

_mesh = plsc.VectorSubcoreMesh(core_axis_name="c", subcore_axis_name="s")
_sc_params = pltpu.CompilerParams(
    needs_layout_passes=False, use_tc_tiling_on_sc=False)


def _sc_pass(table, src2d, dst2d, params):
    @functools.partial(
        pl.kernel,
        out_type=jax.ShapeDtypeStruct((2 * N_NODES, 16), jnp.float32),
        mesh=_mesh,
        compiler_params=_sc_params,
        scratch_types=[
            pltpu.VMEM((CH, LANE), jnp.int32),
            pltpu.VMEM((CH, LANE), jnp.int32),
            pltpu.VMEM((CH, LANE), jnp.int32),
            [pltpu.VMEM((LANE, 16), jnp.float32) for _ in range(NB)],
            pltpu.VMEM((32,), jnp.int32),
            pltpu.VMEM_SHARED((NP, 16), jnp.float32),
            [pltpu.SemaphoreType.DMA for _ in range(NB)],
        ],
    )
    def pass_kernel(table_hbm, src_hbm, dst_hbm, par_hbm, out_hbm,
                    srcb, dstb, gidx, bufs, parb, acc, sems):
        c = lax.axis_index("c")
        s = lax.axis_index("s")
        zero16 = jnp.zeros((16,), jnp.float32)

        pltpu.sync_copy(par_hbm, parb)
        mulv = parb[pl.ds(0, 16)]
        coffv = jnp.broadcast_to(c, (16,)).astype(jnp.int32) * parb[pl.ds(16, 16)]

        def zfill(i, _):
            bufs[0][i] = zero16
            return 0

        lax.fori_loop(0, LANE, zfill, 0)
        base = s * STRIPE

        def zcopy(i, _):
            pltpu.sync_copy(bufs[0], acc.at[pl.ds(base + i * LANE, LANE)])
            return 0

        lax.fori_loop(0, STRIPE // LANE, zcopy, 0)
        pltpu.sync_copy(bufs[0].at[pl.ds(0, STRIPE % LANE)],
                        acc.at[pl.ds(base + (STRIPE // LANE) * LANE,
                                     STRIPE % LANE)])
        plsc.subcore_barrier()

        row0 = s * RPT

        def transform(j):
            for g in range(LANE // 16):
                v = srcb[j, pl.ds(g * 16, 16)]
                gidx[j, pl.ds(g * 16, 16)] = v * mulv + coffv

        for k in range(RPT // CH):
            pltpu.sync_copy(src_hbm.at[pl.ds(row0 + k * CH, CH)], srcb)
            pltpu.sync_copy(dst_hbm.at[pl.ds(row0 + k * CH, CH)], dstb)
            for b in range(NB):
                transform(b)
                pltpu.async_copy(table_hbm.at[gidx.at[b]], bufs[b], sems[b])

            def qstep(q, _):
                for b in range(NB):
                    row = q * NB + b
                    nxt = row + NB

                    @pl.when(nxt < CH)
                    def _():
                        transform(nxt)

                    pltpu.make_async_copy(
                        table_hbm.at[gidx.at[b]], bufs[b], sems[b]).wait()
                    pltpu.sync_copy(bufs[b], acc.at[dstb.at[row]], add=True)

                    @pl.when(nxt < CH)
                    def _():
                        pltpu.async_copy(
                            table_hbm.at[gidx.at[nxt]], bufs[b], sems[b])
                return 0

            lax.fori_loop(0, CH // NB, qstep, 0)

        plsc.subcore_barrier()
        last_valid = N_NODES - 15 * STRIPE

        @pl.when(s != 15)
        def _():
            pltpu.sync_copy(acc.at[pl.ds(base, STRIPE)],
                            out_hbm.at[pl.ds(c * N_NODES + base, STRIPE)])

        @pl.when(s == 15)
        def _():
            pltpu.sync_copy(
                acc.at[pl.ds(15 * STRIPE, last_valid)],
                out_hbm.at[pl.ds(c * N_NODES + 15 * STRIPE, last_valid)])

    return pass_kernel(table, src2d, dst2d, params)


def _tc1_body(deg_ref, x_ref, dinv_ref, g0_ref):
    deg = deg_ref[:, :1] + 1.0
    dinv = lax.rsqrt(deg)
    dinv_ref[...] = dinv
    g0 = x_ref[...] * dinv
    g0_ref[...] = jnp.concatenate(
        [g0, jnp.zeros((BLK, 8), jnp.float32)], axis=-1)


def _tc2_body(s1_ref, g0_ref, dinv_ref, w1_ref, b1_ref, g1_ref):
    dinv = dinv_ref[...]
    a1 = (s1_ref[...] + g0_ref[...]) * dinv
    h1 = jnp.maximum(
        jnp.dot(a1, w1_ref[...], preferred_element_type=jnp.float32)
        + b1_ref[...], 0.0)
    g1_ref[...] = h1 * dinv


def _tc3_body(s2_ref, g1_ref, dinv_ref, w2_ref, b2_ref, wfc_ref, bfc_ref,
              y_ref):
    s2 = jnp.concatenate([s2_ref[0], s2_ref[1]], axis=-1)
    dinv = dinv_ref[...]
    a2 = (s2 + g1_ref[...]) * dinv
    h2 = jnp.maximum(
        jnp.dot(a2, w2_ref[...], preferred_element_type=jnp.float32)
        + b2_ref[...], 0.0)
    y_ref[...] = (
        jnp.dot(h2, wfc_ref[...], preferred_element_type=jnp.float32)
        + bfc_ref[...])


def _full(shape):
    return pl.BlockSpec(shape, lambda i: tuple(0 for _ in shape))


def _row_spec(width):
    return pl.BlockSpec((BLK, width), lambda i: (i, 0))


def _tc1(deg, x):
    return pl.pallas_call(
        _tc1_body,
        grid=(N_NODES // BLK,),
        in_specs=[_row_spec(16), _row_spec(8)],
        out_specs=[_row_spec(1), _row_spec(16)],
        out_shape=[
            jax.ShapeDtypeStruct((N_NODES, 1), jnp.float32),
            jax.ShapeDtypeStruct((2 * N_NODES, 16), jnp.float32),
        ],
    )(deg, x)


def _tc2(s1, g0p, dinv, W1p, b1):
    return pl.pallas_call(
        _tc2_body,
        grid=(N_NODES // BLK,),
        in_specs=[
            _row_spec(16),
            _row_spec(16),
            _row_spec(1),
            _full((16, 32)),
            _full((32,)),
        ],
        out_specs=_row_spec(32),
        out_shape=jax.ShapeDtypeStruct((N_NODES, 32), jnp.float32),
    )(s1, g0p, dinv, W1p, b1)


def _tc3(s2_parts, g1, dinv, W2, b2, Wfc, bfc):
    return pl.pallas_call(
        _tc3_body,
        grid=(N_NODES // BLK,),
        in_specs=[
            pl.BlockSpec((2, BLK, 16), lambda i: (0, i, 0)),
            _row_spec(32),
            _row_spec(1),
            _full((32, 32)),
            _full((32,)),
            _full((32, 1)),
            _full((1,)),
        ],
        out_specs=_row_spec(1),
        out_shape=jax.ShapeDtypeStruct((N_NODES, 1), jnp.float32),
    )(s2_parts, g1, dinv, W2, b2, Wfc, bfc)


def _par(mul, cmul):
    return jnp.concatenate([
        jnp.full((16,), mul, jnp.int32),
        jnp.full((16,), cmul, jnp.int32),
    ])


@jax.jit
def kernel(x, edge_index, W1, b1, W2, b2, Wfc, bfc):
    pad = E_PAD - N_EDGES
    srcp = jnp.concatenate(
        [edge_index[0], jnp.zeros((pad,), jnp.int32)]).reshape(ROWS, LANE)
    dstp = jnp.concatenate(
        [edge_index[1], jnp.full((pad,), N_NODES, jnp.int32)]).reshape(
            ROWS, LANE)

    ones_t = jnp.ones((2 * N_NODES, 16), jnp.float32)
    deg_parts = _sc_pass(ones_t, srcp, dstp, _par(1, 0))
    dinv, g0p = _tc1(deg_parts[:N_NODES], x)

    s1 = _sc_pass(g0p, srcp, dstp, _par(1, 0))
    W1p = jnp.concatenate([W1, jnp.zeros((8, 32), jnp.float32)], axis=0)
    g1 = _tc2(s1[:N_NODES], g0p[:N_NODES], dinv, W1p, b1)

    s2 = _sc_pass(g1.reshape(2 * N_NODES, 16), srcp, dstp, _par(2, 1))
    return _tc3(s2.reshape(2, N_NODES, 16), g1, dinv, W2, b2, Wfc, bfc)


# speedup vs baseline: 19.3288x; 1.0536x over previous
import functools

import jax
import jax.numpy as jnp
from jax import lax
from jax.experimental import pallas as pl
from jax.experimental.pallas import tpu as pltpu
from jax.experimental.pallas import tpu_sc as plsc

N_NODES = 100000
N_EDGES = 1600000
BLK = 4000
LANE = 128
E_PAD = 1638400
ROWS = E_PAD // LANE
NTR = 96
NP = N_NODES + NTR
STRIPE = NP // 16
CH = 40
NB = 8
RPT = ROWS // 16

_mesh = plsc.VectorSubcoreMesh(core_axis_name="c", subcore_axis_name="s")
_sc_params = pltpu.CompilerParams(
    needs_layout_passes=False, use_tc_tiling_on_sc=False)


def _sc_pass(table, src2d, dst2d, params):
    @functools.partial(
        pl.kernel,
        out_type=jax.ShapeDtypeStruct((2 * N_NODES, 16), jnp.float32),
        mesh=_mesh,
        compiler_params=_sc_params,
        scratch_types=[
            pltpu.VMEM((CH, LANE), jnp.int32),
            pltpu.VMEM((CH, LANE), jnp.int32),
            [pltpu.VMEM((LANE, 16), jnp.float32) for _ in range(NB)],
            pltpu.VMEM((32,), jnp.int32),
            pltpu.VMEM_SHARED((NP, 16), jnp.float32),
            [pltpu.SemaphoreType.DMA for _ in range(NB)],
        ],
    )
    def pass_kernel(table_hbm, src_hbm, dst_hbm, par_hbm, out_hbm,
                    srcb, dstb, bufs, parb, acc, sems):
        c = lax.axis_index("c")
        s = lax.axis_index("s")
        zero16 = jnp.zeros((16,), jnp.float32)

        pltpu.sync_copy(par_hbm, parb)
        mulv = parb[pl.ds(0, 16)]
        coffv = jnp.broadcast_to(c, (16,)).astype(jnp.int32) * parb[pl.ds(16, 16)]

        def zfill(i, _):
            bufs[0][i] = zero16
            return 0

        lax.fori_loop(0, LANE, zfill, 0)
        base = s * STRIPE

        def zcopy(i, _):
            pltpu.sync_copy(bufs[0], acc.at[pl.ds(base + i * LANE, LANE)])
            return 0

        lax.fori_loop(0, STRIPE // LANE, zcopy, 0)
        pltpu.sync_copy(bufs[0].at[pl.ds(0, STRIPE % LANE)],
                        acc.at[pl.ds(base + (STRIPE // LANE) * LANE,
                                     STRIPE % LANE)])
        plsc.subcore_barrier()

        row0 = s * RPT

        def transform(j):
            for g in range(LANE // 16):
                v = srcb[j, pl.ds(g * 16, 16)]
                srcb[j, pl.ds(g * 16, 16)] = v * mulv + coffv

        for k in range(RPT // CH):
            pltpu.sync_copy(src_hbm.at[pl.ds(row0 + k * CH, CH)], srcb)
            pltpu.sync_copy(dst_hbm.at[pl.ds(row0 + k * CH, CH)], dstb)
            for b in range(NB):
                transform(b)
                pltpu.async_copy(table_hbm.at[srcb.at[b]], bufs[b], sems[b])

            def qstep(q, _):
                for b in range(NB):
                    row = q * NB + b
                    nxt = row + NB

                    @pl.when(nxt < CH)
                    def _():
                        transform(nxt)

                    pltpu.make_async_copy(
                        table_hbm.at[srcb.at[b]], bufs[b], sems[b]).wait()
                    pltpu.sync_copy(bufs[b], acc.at[dstb.at[row]], add=True)

                    @pl.when(nxt < CH)
                    def _():
                        pltpu.async_copy(
                            table_hbm.at[srcb.at[nxt]], bufs[b], sems[b])
                return 0

            lax.fori_loop(0, CH // NB, qstep, 0)

        plsc.subcore_barrier()
        last_valid = N_NODES - 15 * STRIPE

        @pl.when(s != 15)
        def _():
            pltpu.sync_copy(acc.at[pl.ds(base, STRIPE)],
                            out_hbm.at[pl.ds(c * N_NODES + base, STRIPE)])

        @pl.when(s == 15)
        def _():
            pltpu.sync_copy(
                acc.at[pl.ds(15 * STRIPE, last_valid)],
                out_hbm.at[pl.ds(c * N_NODES + 15 * STRIPE, last_valid)])

    return pass_kernel(table, src2d, dst2d, params)


def _tc1_body(deg_ref, x_ref, dinv_ref, g0_ref):
    deg = deg_ref[:, :1] + 1.0
    dinv = lax.rsqrt(deg)
    dinv_ref[...] = dinv
    g0 = x_ref[...] * dinv
    g0_ref[...] = jnp.concatenate(
        [g0, jnp.zeros((BLK, 8), jnp.float32)], axis=-1)


def _tc2_body(s1_ref, g0_ref, dinv_ref, w1_ref, b1_ref, g1_ref):
    dinv = dinv_ref[...]
    a1 = (s1_ref[...] + g0_ref[...]) * dinv
    h1 = jnp.maximum(
        jnp.dot(a1, w1_ref[...], preferred_element_type=jnp.float32)
        + b1_ref[...], 0.0)
    g1_ref[...] = h1 * dinv


def _tc3_body(s2_ref, g1_ref, dinv_ref, w2_ref, b2_ref, wfc_ref, bfc_ref,
              y_ref):
    s2 = jnp.concatenate([s2_ref[0], s2_ref[1]], axis=-1)
    dinv = dinv_ref[...]
    a2 = (s2 + g1_ref[...]) * dinv
    h2 = jnp.maximum(
        jnp.dot(a2, w2_ref[...], preferred_element_type=jnp.float32)
        + b2_ref[...], 0.0)
    y_ref[...] = (
        jnp.dot(h2, wfc_ref[...], preferred_element_type=jnp.float32)
        + bfc_ref[...])


def _full(shape):
    return pl.BlockSpec(shape, lambda i: tuple(0 for _ in shape))


def _row_spec(width):
    return pl.BlockSpec((BLK, width), lambda i: (i, 0))


def _tc1(deg, x):
    return pl.pallas_call(
        _tc1_body,
        grid=(N_NODES // BLK,),
        in_specs=[_row_spec(16), _row_spec(8)],
        out_specs=[_row_spec(1), _row_spec(16)],
        out_shape=[
            jax.ShapeDtypeStruct((N_NODES, 1), jnp.float32),
            jax.ShapeDtypeStruct((2 * N_NODES, 16), jnp.float32),
        ],
    )(deg, x)


def _tc2(s1, g0p, dinv, W1p, b1):
    return pl.pallas_call(
        _tc2_body,
        grid=(N_NODES // BLK,),
        in_specs=[
            _row_spec(16),
            _row_spec(16),
            _row_spec(1),
            _full((16, 32)),
            _full((32,)),
        ],
        out_specs=_row_spec(32),
        out_shape=jax.ShapeDtypeStruct((N_NODES, 32), jnp.float32),
    )(s1, g0p, dinv, W1p, b1)


def _tc3(s2_parts, g1, dinv, W2, b2, Wfc, bfc):
    return pl.pallas_call(
        _tc3_body,
        grid=(N_NODES // BLK,),
        in_specs=[
            pl.BlockSpec((2, BLK, 16), lambda i: (0, i, 0)),
            _row_spec(32),
            _row_spec(1),
            _full((32, 32)),
            _full((32,)),
            _full((32, 1)),
            _full((1,)),
        ],
        out_specs=_row_spec(1),
        out_shape=jax.ShapeDtypeStruct((N_NODES, 1), jnp.float32),
    )(s2_parts, g1, dinv, W2, b2, Wfc, bfc)


def _par(mul, cmul):
    return jnp.concatenate([
        jnp.full((16,), mul, jnp.int32),
        jnp.full((16,), cmul, jnp.int32),
    ])


@jax.jit
def kernel(x, edge_index, W1, b1, W2, b2, Wfc, bfc):
    pad = E_PAD - N_EDGES
    srcp = jnp.concatenate(
        [edge_index[0], jnp.zeros((pad,), jnp.int32)]).reshape(ROWS, LANE)
    dstp = jnp.concatenate(
        [edge_index[1], jnp.full((pad,), N_NODES, jnp.int32)]).reshape(
            ROWS, LANE)

    ones_t = jnp.ones((2 * N_NODES, 16), jnp.float32)
    deg_parts = _sc_pass(ones_t, srcp, dstp, _par(1, 0))
    dinv, g0p = _tc1(deg_parts[:N_NODES], x)

    s1 = _sc_pass(g0p, srcp, dstp, _par(1, 0))
    W1p = jnp.concatenate([W1, jnp.zeros((8, 32), jnp.float32)], axis=0)
    g1 = _tc2(s1[:N_NODES], g0p[:N_NODES], dinv, W1p, b1)

    s2 = _sc_pass(g1.reshape(2 * N_NODES, 16), srcp, dstp, _par(2, 1))
    return _tc3(s2.reshape(2, N_NODES, 16), g1, dinv, W2, b2, Wfc, bfc)


# speedup vs baseline: 23.1815x; 1.1993x over previous
import functools

import jax
import jax.numpy as jnp
from jax import lax
from jax.experimental import pallas as pl
from jax.experimental.pallas import tpu as pltpu
from jax.experimental.pallas import tpu_sc as plsc

N_NODES = 100000
N_EDGES = 1600000
BLK = 4000
LANE = 128
E_PAD = 1638400
ROWS = E_PAD // LANE
NTR = 96
NP = N_NODES + NTR
STRIPE = NP // 16
CH = 40
NB = 10
KD = 5
RPT = ROWS // 16

_mesh = plsc.VectorSubcoreMesh(core_axis_name="c", subcore_axis_name="s")
_sc_params = pltpu.CompilerParams(
    needs_layout_passes=False, use_tc_tiling_on_sc=False)


def _sc_pass(table, src2d, dst2d, params):
    @functools.partial(
        pl.kernel,
        out_type=jax.ShapeDtypeStruct((2 * N_NODES, 16), jnp.float32),
        mesh=_mesh,
        compiler_params=_sc_params,
        scratch_types=[
            pltpu.VMEM((CH, LANE), jnp.int32),
            pltpu.VMEM((CH, LANE), jnp.int32),
            [pltpu.VMEM((LANE, 16), jnp.float32) for _ in range(NB)],
            pltpu.VMEM((48,), jnp.int32),
            pltpu.VMEM_SHARED((NP, 16), jnp.float32),
            [pltpu.SemaphoreType.DMA for _ in range(NB)],
            [pltpu.SemaphoreType.DMA for _ in range(NB)],
        ],
    )
    def pass_kernel(table_hbm, src_hbm, dst_hbm, par_hbm, out_hbm,
                    srcb, dstb, bufs, parb, acc, gsems, ssems):
        c = lax.axis_index("c")
        s = lax.axis_index("s")
        zero16 = jnp.zeros((16,), jnp.float32)

        pltpu.sync_copy(par_hbm, parb)
        mulv = parb[pl.ds(0, 16)]
        coffv = jnp.broadcast_to(c, (16,)).astype(jnp.int32) * parb[pl.ds(16, 16)]
        iota16 = lax.iota(jnp.int32, 16)
        pv3 = parb[pl.ds(32, 16)]
        neg = jnp.full((16,), -(2**31) + 1, jnp.int32)
        rows_a = jnp.max(jnp.where(iota16 == 0, pv3, neg))
        rows_b = jnp.max(jnp.where(iota16 == 1, pv3, neg))
        nchunks = jnp.max(jnp.where(iota16 == 2, pv3, neg))

        def zfill(i, _):
            bufs[0][i] = zero16
            return 0

        lax.fori_loop(0, LANE, zfill, 0)
        base = s * STRIPE

        def zcopy(i, _):
            pltpu.sync_copy(bufs[0], acc.at[pl.ds(base + i * LANE, LANE)])
            return 0

        lax.fori_loop(0, STRIPE // LANE, zcopy, 0)
        pltpu.sync_copy(bufs[0].at[pl.ds(0, STRIPE % LANE)],
                        acc.at[pl.ds(base + (STRIPE // LANE) * LANE,
                                     STRIPE % LANE)])
        plsc.subcore_barrier()

        row0 = s * rows_a + c * rows_b

        def transform(j):
            for g in range(LANE // 16):
                v = srcb[j, pl.ds(g * 16, 16)]
                srcb[j, pl.ds(g * 16, 16)] = v * mulv + coffv

        def chunk_body(k, _):
            pltpu.sync_copy(src_hbm.at[pl.ds(row0 + k * CH, CH)], srcb)
            pltpu.sync_copy(dst_hbm.at[pl.ds(row0 + k * CH, CH)], dstb)
            for b in range(NB):
                transform(b)
                pltpu.async_copy(table_hbm.at[srcb.at[b]], bufs[b], gsems[b])

            def qstep(q, _):
                for b in range(NB):
                    row = q * NB + b
                    nxt = row + NB

                    @pl.when(nxt < CH)
                    def _():
                        transform(nxt)

                    pltpu.make_async_copy(
                        table_hbm.at[srcb.at[b]], bufs[b], gsems[b]).wait()
                    pltpu.async_copy(bufs[b], acc.at[dstb.at[row]],
                                     ssems[b], add=True)

                    prow = row - KD
                    pb = (b - KD) % NB
                    pnxt = prow + NB

                    @pl.when(prow >= 0)
                    def _():
                        pltpu.make_async_copy(
                            bufs[pb], acc.at[dstb.at[prow]], ssems[pb]).wait()

                    @pl.when(jnp.logical_and(prow >= 0, pnxt < CH))
                    def _():
                        pltpu.async_copy(
                            table_hbm.at[srcb.at[pnxt]], bufs[pb], gsems[pb])
                return 0

            lax.fori_loop(0, CH // NB, qstep, 0)
            for r in range(CH - KD, CH):
                pltpu.make_async_copy(
                    bufs[r % NB], acc.at[dstb.at[r]], ssems[r % NB]).wait()
            return 0

        lax.fori_loop(0, nchunks, chunk_body, 0)

        plsc.subcore_barrier()
        last_valid = N_NODES - 15 * STRIPE

        @pl.when(s != 15)
        def _():
            pltpu.sync_copy(acc.at[pl.ds(base, STRIPE)],
                            out_hbm.at[pl.ds(c * N_NODES + base, STRIPE)])

        @pl.when(s == 15)
        def _():
            pltpu.sync_copy(
                acc.at[pl.ds(15 * STRIPE, last_valid)],
                out_hbm.at[pl.ds(c * N_NODES + 15 * STRIPE, last_valid)])

    return pass_kernel(table, src2d, dst2d, params)


def _tc1_body(deg_ref, deg1_ref, x_ref, dinv_ref, g0_ref):
    deg = deg_ref[:, :1] + deg1_ref[:, :1] + 1.0
    dinv = lax.rsqrt(deg)
    dinv_ref[...] = dinv
    g0 = x_ref[...] * dinv
    g0_ref[...] = jnp.concatenate(
        [g0, jnp.zeros((BLK, 8), jnp.float32)], axis=-1)


def _tc2_body(s1_ref, s1b_ref, g0_ref, dinv_ref, w1_ref, b1_ref, g1_ref):
    dinv = dinv_ref[...]
    a1 = (s1_ref[...] + s1b_ref[...] + g0_ref[...]) * dinv
    h1 = jnp.maximum(
        jnp.dot(a1, w1_ref[...], preferred_element_type=jnp.float32)
        + b1_ref[...], 0.0)
    g1_ref[...] = h1 * dinv


def _tc3_body(s2_ref, g1_ref, dinv_ref, w2_ref, b2_ref, wfc_ref, bfc_ref,
              y_ref):
    s2 = jnp.concatenate([s2_ref[0], s2_ref[1]], axis=-1)
    dinv = dinv_ref[...]
    a2 = (s2 + g1_ref[...]) * dinv
    h2 = jnp.maximum(
        jnp.dot(a2, w2_ref[...], preferred_element_type=jnp.float32)
        + b2_ref[...], 0.0)
    y_ref[...] = (
        jnp.dot(h2, wfc_ref[...], preferred_element_type=jnp.float32)
        + bfc_ref[...])


def _full(shape):
    return pl.BlockSpec(shape, lambda i: tuple(0 for _ in shape))


def _row_spec(width):
    return pl.BlockSpec((BLK, width), lambda i: (i, 0))


def _tc1(deg, deg1, x):
    return pl.pallas_call(
        _tc1_body,
        grid=(N_NODES // BLK,),
        in_specs=[_row_spec(16), _row_spec(16), _row_spec(8)],
        out_specs=[_row_spec(1), _row_spec(16)],
        out_shape=[
            jax.ShapeDtypeStruct((N_NODES, 1), jnp.float32),
            jax.ShapeDtypeStruct((2 * N_NODES, 16), jnp.float32),
        ],
    )(deg, deg1, x)


def _tc2(s1, s1b, g0p, dinv, W1p, b1):
    return pl.pallas_call(
        _tc2_body,
        grid=(N_NODES // BLK,),
        in_specs=[
            _row_spec(16),
            _row_spec(16),
            _row_spec(16),
            _row_spec(1),
            _full((16, 32)),
            _full((32,)),
        ],
        out_specs=_row_spec(32),
        out_shape=jax.ShapeDtypeStruct((N_NODES, 32), jnp.float32),
    )(s1, s1b, g0p, dinv, W1p, b1)


def _tc3(s2_parts, g1, dinv, W2, b2, Wfc, bfc):
    return pl.pallas_call(
        _tc3_body,
        grid=(N_NODES // BLK,),
        in_specs=[
            pl.BlockSpec((2, BLK, 16), lambda i: (0, i, 0)),
            _row_spec(32),
            _row_spec(1),
            _full((32, 32)),
            _full((32,)),
            _full((32, 1)),
            _full((1,)),
        ],
        out_specs=_row_spec(1),
        out_shape=jax.ShapeDtypeStruct((N_NODES, 1), jnp.float32),
    )(s2_parts, g1, dinv, W2, b2, Wfc, bfc)


def _par(mul, cmul, rows_a, rows_b, nchunks):
    tail = jnp.zeros((16,), jnp.int32)
    tail = tail.at[0].set(rows_a).at[1].set(rows_b).at[2].set(nchunks)
    return jnp.concatenate([
        jnp.full((16,), mul, jnp.int32),
        jnp.full((16,), cmul, jnp.int32),
        tail,
    ])


@jax.jit
def kernel(x, edge_index, W1, b1, W2, b2, Wfc, bfc):
    pad = E_PAD - N_EDGES
    srcp = jnp.concatenate(
        [edge_index[0], jnp.zeros((pad,), jnp.int32)]).reshape(ROWS, LANE)
    dstp = jnp.concatenate(
        [edge_index[1], jnp.full((pad,), N_NODES, jnp.int32)]).reshape(
            ROWS, LANE)

    split = _par(1, 0, ROWS // 32, ROWS // 2, ROWS // 32 // CH)
    full = _par(2, 1, ROWS // 16, 0, ROWS // 16 // CH)

    ones_t = jnp.ones((2 * N_NODES, 16), jnp.float32)
    deg_parts = _sc_pass(ones_t, srcp, dstp, split)
    dinv, g0p = _tc1(deg_parts[:N_NODES], deg_parts[N_NODES:], x)

    s1 = _sc_pass(g0p, srcp, dstp, split)
    W1p = jnp.concatenate([W1, jnp.zeros((8, 32), jnp.float32)], axis=0)
    g1 = _tc2(s1[:N_NODES], s1[N_NODES:], g0p[:N_NODES], dinv, W1p, b1)

    s2 = _sc_pass(g1.reshape(2 * N_NODES, 16), srcp, dstp, full)
    return _tc3(s2.reshape(2, N_NODES, 16), g1, dinv, W2, b2, Wfc, bfc)


# speedup vs baseline: 23.8455x; 1.0286x over previous
import functools

import jax
import jax.numpy as jnp
from jax import lax
from jax.experimental import pallas as pl
from jax.experimental.pallas import tpu as pltpu
from jax.experimental.pallas import tpu_sc as plsc

N_NODES = 100000
N_EDGES = 1600000
BLK = 4000
LANE = 128
E_PAD = 1638400
ROWS = E_PAD // LANE
NTR = 96
NP = N_NODES + NTR
STRIPE = NP // 16
CH = 40
NB = 10
KD = 3
RPT = ROWS // 16

_mesh = plsc.VectorSubcoreMesh(core_axis_name="c", subcore_axis_name="s")
_sc_params = pltpu.CompilerParams(
    needs_layout_passes=False, use_tc_tiling_on_sc=False)


def _sc_pass(table, src2d, dst2d, params):
    @functools.partial(
        pl.kernel,
        out_type=jax.ShapeDtypeStruct((2 * N_NODES, 16), jnp.float32),
        mesh=_mesh,
        compiler_params=_sc_params,
        scratch_types=[
            pltpu.VMEM((CH, LANE), jnp.int32),
            pltpu.VMEM((CH, LANE), jnp.int32),
            [pltpu.VMEM((LANE, 16), jnp.float32) for _ in range(NB)],
            pltpu.VMEM((48,), jnp.int32),
            pltpu.VMEM_SHARED((NP, 16), jnp.float32),
            [pltpu.SemaphoreType.DMA for _ in range(NB)],
            [pltpu.SemaphoreType.DMA for _ in range(NB)],
        ],
    )
    def pass_kernel(table_hbm, src_hbm, dst_hbm, par_hbm, out_hbm,
                    srcb, dstb, bufs, parb, acc, gsems, ssems):
        c = lax.axis_index("c")
        s = lax.axis_index("s")
        zero16 = jnp.zeros((16,), jnp.float32)

        pltpu.sync_copy(par_hbm, parb)
        mulv = parb[pl.ds(0, 16)]
        coffv = jnp.broadcast_to(c, (16,)).astype(jnp.int32) * parb[pl.ds(16, 16)]
        iota16 = lax.iota(jnp.int32, 16)
        pv3 = parb[pl.ds(32, 16)]
        neg = jnp.full((16,), -(2**31) + 1, jnp.int32)
        rows_a = jnp.max(jnp.where(iota16 == 0, pv3, neg))
        rows_b = jnp.max(jnp.where(iota16 == 1, pv3, neg))
        nchunks = jnp.max(jnp.where(iota16 == 2, pv3, neg))

        def zfill(i, _):
            bufs[0][i] = zero16
            return 0

        lax.fori_loop(0, LANE, zfill, 0)
        base = s * STRIPE

        def zcopy(i, _):
            pltpu.sync_copy(bufs[0], acc.at[pl.ds(base + i * LANE, LANE)])
            return 0

        lax.fori_loop(0, STRIPE // LANE, zcopy, 0)
        pltpu.sync_copy(bufs[0].at[pl.ds(0, STRIPE % LANE)],
                        acc.at[pl.ds(base + (STRIPE // LANE) * LANE,
                                     STRIPE % LANE)])
        plsc.subcore_barrier()

        row0 = s * rows_a + c * rows_b

        def transform(j):
            for g in range(LANE // 16):
                v = srcb[j, pl.ds(g * 16, 16)]
                srcb[j, pl.ds(g * 16, 16)] = v * mulv + coffv

        def chunk_body(k, _):
            pltpu.sync_copy(src_hbm.at[pl.ds(row0 + k * CH, CH)], srcb)
            pltpu.sync_copy(dst_hbm.at[pl.ds(row0 + k * CH, CH)], dstb)
            for b in range(NB):
                transform(b)
                pltpu.async_copy(table_hbm.at[srcb.at[b]], bufs[b], gsems[b])

            def qstep(q, _):
                for b in range(NB):
                    row = q * NB + b
                    nxt = row + NB

                    @pl.when(nxt < CH)
                    def _():
                        transform(nxt)

                    pltpu.make_async_copy(
                        table_hbm.at[srcb.at[b]], bufs[b], gsems[b]).wait()
                    pltpu.async_copy(bufs[b], acc.at[dstb.at[row]],
                                     ssems[b], add=True)

                    prow = row - KD
                    pb = (b - KD) % NB
                    pnxt = prow + NB

                    @pl.when(prow >= 0)
                    def _():
                        pltpu.make_async_copy(
                            bufs[pb], acc.at[dstb.at[prow]], ssems[pb]).wait()

                    @pl.when(jnp.logical_and(prow >= 0, pnxt < CH))
                    def _():
                        pltpu.async_copy(
                            table_hbm.at[srcb.at[pnxt]], bufs[pb], gsems[pb])
                return 0

            lax.fori_loop(0, CH // NB, qstep, 0)
            for r in range(CH - KD, CH):
                pltpu.make_async_copy(
                    bufs[r % NB], acc.at[dstb.at[r]], ssems[r % NB]).wait()
            return 0

        lax.fori_loop(0, nchunks, chunk_body, 0)

        plsc.subcore_barrier()
        last_valid = N_NODES - 15 * STRIPE

        @pl.when(s != 15)
        def _():
            pltpu.sync_copy(acc.at[pl.ds(base, STRIPE)],
                            out_hbm.at[pl.ds(c * N_NODES + base, STRIPE)])

        @pl.when(s == 15)
        def _():
            pltpu.sync_copy(
                acc.at[pl.ds(15 * STRIPE, last_valid)],
                out_hbm.at[pl.ds(c * N_NODES + 15 * STRIPE, last_valid)])

    return pass_kernel(table, src2d, dst2d, params)


def _tc1_body(deg_ref, deg1_ref, x_ref, dinv_ref, g0_ref):
    deg = deg_ref[:, :1] + deg1_ref[:, :1] + 1.0
    dinv = lax.rsqrt(deg)
    dinv_ref[...] = dinv
    g0 = x_ref[...] * dinv
    g0_ref[...] = jnp.concatenate(
        [g0, jnp.zeros((BLK, 8), jnp.float32)], axis=-1)


def _tc2_body(s1_ref, s1b_ref, g0_ref, dinv_ref, w1_ref, b1_ref, g1_ref):
    dinv = dinv_ref[...]
    a1 = (s1_ref[...] + s1b_ref[...] + g0_ref[...]) * dinv
    h1 = jnp.maximum(
        jnp.dot(a1, w1_ref[...], preferred_element_type=jnp.float32)
        + b1_ref[...], 0.0)
    g1_ref[...] = h1 * dinv


def _tc3_body(s2_ref, g1_ref, dinv_ref, w2_ref, b2_ref, wfc_ref, bfc_ref,
              y_ref):
    s2 = jnp.concatenate([s2_ref[0], s2_ref[1]], axis=-1)
    dinv = dinv_ref[...]
    a2 = (s2 + g1_ref[...]) * dinv
    h2 = jnp.maximum(
        jnp.dot(a2, w2_ref[...], preferred_element_type=jnp.float32)
        + b2_ref[...], 0.0)
    y_ref[...] = (
        jnp.dot(h2, wfc_ref[...], preferred_element_type=jnp.float32)
        + bfc_ref[...])


def _full(shape):
    return pl.BlockSpec(shape, lambda i: tuple(0 for _ in shape))


def _row_spec(width):
    return pl.BlockSpec((BLK, width), lambda i: (i, 0))


def _tc1(deg_parts, x):
    nb = N_NODES // BLK
    return pl.pallas_call(
        _tc1_body,
        grid=(nb,),
        in_specs=[
            pl.BlockSpec((BLK, 16), lambda i: (i, 0)),
            pl.BlockSpec((BLK, 16), lambda i: (i + N_NODES // BLK, 0)),
            _row_spec(8),
        ],
        out_specs=[_row_spec(1), _row_spec(16)],
        out_shape=[
            jax.ShapeDtypeStruct((N_NODES, 1), jnp.float32),
            jax.ShapeDtypeStruct((2 * N_NODES, 16), jnp.float32),
        ],
    )(deg_parts, deg_parts, x)


def _tc2(s1, g0p, dinv, W1p, b1):
    return pl.pallas_call(
        _tc2_body,
        grid=(N_NODES // BLK,),
        in_specs=[
            pl.BlockSpec((BLK, 16), lambda i: (i, 0)),
            pl.BlockSpec((BLK, 16), lambda i: (i + N_NODES // BLK, 0)),
            _row_spec(16),
            _row_spec(1),
            _full((16, 32)),
            _full((32,)),
        ],
        out_specs=_row_spec(32),
        out_shape=jax.ShapeDtypeStruct((N_NODES, 32), jnp.float32),
    )(s1, s1, g0p, dinv, W1p, b1)


def _tc3(s2_parts, g1, dinv, W2, b2, Wfc, bfc):
    return pl.pallas_call(
        _tc3_body,
        grid=(N_NODES // BLK,),
        in_specs=[
            pl.BlockSpec((2, BLK, 16), lambda i: (0, i, 0)),
            _row_spec(32),
            _row_spec(1),
            _full((32, 32)),
            _full((32,)),
            _full((32, 1)),
            _full((1,)),
        ],
        out_specs=_row_spec(1),
        out_shape=jax.ShapeDtypeStruct((N_NODES, 1), jnp.float32),
    )(s2_parts, g1, dinv, W2, b2, Wfc, bfc)


def _par(mul, cmul, rows_a, rows_b, nchunks):
    tail = jnp.zeros((16,), jnp.int32)
    tail = tail.at[0].set(rows_a).at[1].set(rows_b).at[2].set(nchunks)
    return jnp.concatenate([
        jnp.full((16,), mul, jnp.int32),
        jnp.full((16,), cmul, jnp.int32),
        tail,
    ])


@jax.jit
def kernel(x, edge_index, W1, b1, W2, b2, Wfc, bfc):
    pad = E_PAD - N_EDGES
    srcp = jnp.concatenate(
        [edge_index[0], jnp.zeros((pad,), jnp.int32)]).reshape(ROWS, LANE)
    dstp = jnp.concatenate(
        [edge_index[1], jnp.full((pad,), N_NODES, jnp.int32)]).reshape(
            ROWS, LANE)

    split = _par(1, 0, ROWS // 32, ROWS // 2, ROWS // 32 // CH)
    full = _par(2, 1, ROWS // 16, 0, ROWS // 16 // CH)

    ones_t = jnp.ones((2 * N_NODES, 16), jnp.float32)
    deg_parts = _sc_pass(ones_t, srcp, dstp, split)
    dinv, g0p = _tc1(deg_parts, x)

    s1 = _sc_pass(g0p, srcp, dstp, split)
    W1p = jnp.concatenate([W1, jnp.zeros((8, 32), jnp.float32)], axis=0)
    g1 = _tc2(s1, g0p, dinv, W1p, b1)

    s2 = _sc_pass(g1.reshape(2 * N_NODES, 16), srcp, dstp, full)
    return _tc3(s2.reshape(2, N_NODES, 16), g1, dinv, W2, b2, Wfc, bfc)
